# Initial kernel scaffold; baseline (speedup 1.0000x reference)
import jax
import jax.numpy as jnp
from jax.experimental import pallas as pl


def kernel(features, price, adj, train_set, W_cid2, b_cid2, W_cid3, b_cid3, emb_price, W_emb, b_emb, W_low, W_mid, g1, be1, g2, be2, W_cat, b_cat):
    raise NotImplementedError("write your pallas kernel here")


# speedup vs baseline: 1.0813x; 1.0813x over previous
import functools

import jax
import jax.numpy as jnp
from jax import lax
from jax.experimental import pallas as pl
from jax.experimental.pallas import tpu as pltpu
from jax.experimental.pallas import tpu_sc as plsc

N = 10000
EMB = 128
CAT = 768
NBINS = 100
BSZ = 1024
NNEG = 4

RB = 400
NB = N // RB
NGATHER = BSZ * (2 + NNEG)

_F32 = jnp.float32


def _item_body(f_ref, p_ref, wc2_ref, wc3_ref, we_ref, bc2_ref, bc3_ref,
               be_ref, ep_ref, item_ref, s_ref, wf_s, pf_s, b_s, acc_s):
    i = pl.program_id(0)

    @pl.when(i == 0)
    def _():
        we1 = we_ref[0:EMB, :]
        we2 = we_ref[EMB:2 * EMB, :]
        we3 = we_ref[2 * EMB:3 * EMB, :]
        wf_s[0:CAT, :] = jnp.dot(wc2_ref[...], we1, preferred_element_type=_F32)
        wf_s[CAT:2 * CAT, :] = jnp.dot(wc3_ref[...], we2, preferred_element_type=_F32)
        pf_s[...] = jnp.dot(ep_ref[...], we3, preferred_element_type=_F32)
        b_s[...] = (jnp.dot(bc2_ref[...], we1, preferred_element_type=_F32)
                    + jnp.dot(bc3_ref[...], we2, preferred_element_type=_F32)
                    + be_ref[...])
        acc_s[...] = jnp.zeros_like(acc_s)

    onehot = (p_ref[...] == lax.broadcasted_iota(jnp.int32, (RB, 128), 1)).astype(_F32)
    x = jnp.dot(f_ref[...], wf_s[...], preferred_element_type=_F32)
    x = x + jnp.dot(onehot, pf_s[...], preferred_element_type=_F32) + b_s[...]
    it = jnp.maximum(x, 0.0)
    item_ref[...] = it
    acc_s[...] += jnp.sum(it, axis=0, keepdims=True)

    @pl.when(i == NB - 1)
    def _():
        s_ref[...] = acc_s[...]


def _item_call(features, pricef, wc2, wc3, we, bc2, bc3, be, ep_pad):
    return pl.pallas_call(
        _item_body,
        grid=(NB,),
        in_specs=[
            pl.BlockSpec((RB, 2 * CAT), lambda i: (i, 0)),
            pl.BlockSpec((RB, 1), lambda i: (i, 0)),
            pl.BlockSpec((CAT, EMB), lambda i: (0, 0)),
            pl.BlockSpec((CAT, EMB), lambda i: (0, 0)),
            pl.BlockSpec((3 * EMB, EMB), lambda i: (0, 0)),
            pl.BlockSpec((1, EMB), lambda i: (0, 0)),
            pl.BlockSpec((1, EMB), lambda i: (0, 0)),
            pl.BlockSpec((1, EMB), lambda i: (0, 0)),
            pl.BlockSpec((128, EMB), lambda i: (0, 0)),
        ],
        out_specs=[
            pl.BlockSpec((RB, EMB), lambda i: (i, 0)),
            pl.BlockSpec((1, EMB), lambda i: (0, 0)),
        ],
        out_shape=[
            jax.ShapeDtypeStruct((N, EMB), _F32),
            jax.ShapeDtypeStruct((1, EMB), _F32),
        ],
        scratch_shapes=[
            pltpu.VMEM((2 * CAT, EMB), _F32),
            pltpu.VMEM((128, EMB), _F32),
            pltpu.VMEM((1, EMB), _F32),
            pltpu.VMEM((1, EMB), _F32),
        ],
    )(features, pricef, wc2, wc3, we, bc2, bc3, be, ep_pad)


def _spmm_body(adj_ref, x_ref, t_ref, s_ref, acc_s):
    i = pl.program_id(0)

    @pl.when(i == 0)
    def _():
        acc_s[...] = jnp.zeros_like(acc_s)

    t = jnp.dot(adj_ref[...], x_ref[...], preferred_element_type=_F32)
    t_ref[...] = t
    acc_s[...] += jnp.sum(t, axis=0, keepdims=True)

    @pl.when(i == NB - 1)
    def _():
        s_ref[...] = acc_s[...]


def _spmm_call(adj, x):
    return pl.pallas_call(
        _spmm_body,
        grid=(NB,),
        in_specs=[
            pl.BlockSpec((RB, N), lambda i: (i, 0)),
            pl.BlockSpec((N, EMB), lambda i: (0, 0)),
        ],
        out_specs=[
            pl.BlockSpec((RB, EMB), lambda i: (i, 0)),
            pl.BlockSpec((1, EMB), lambda i: (0, 0)),
        ],
        out_shape=[
            jax.ShapeDtypeStruct((N, EMB), _F32),
            jax.ShapeDtypeStruct((1, EMB), _F32),
        ],
        scratch_shapes=[pltpu.VMEM((1, EMB), _F32)],
    )(adj, x)


def _gram_body(t1_ref, t2_ref, it_ref, s1_ref, s2_ref, si_ref,
               glp_ref, gmp_ref, mu_s, glp_s, gmp_s):
    i = pl.program_id(0)

    @pl.when(i == 0)
    def _():
        ninv = 1.0 / N
        mu_s[0:1, :] = 0.5 * (s1_ref[...] + si_ref[...]) * ninv
        mu_s[1:2, :] = 0.5 * (s2_ref[...] - si_ref[...]) * ninv
        glp_s[...] = jnp.zeros_like(glp_s)
        gmp_s[...] = jnp.zeros_like(gmp_s)

    t1 = t1_ref[...]
    t2 = t2_ref[...]
    it = it_ref[...]
    lpc = 0.5 * (t1 + it) - mu_s[0:1, :]
    mpc = 0.5 * (t2 - it) - mu_s[1:2, :]
    dn = (((0,), (0,)), ((), ()))
    glp_s[...] += lax.dot_general(lpc, lpc, dn, preferred_element_type=_F32)
    gmp_s[...] += lax.dot_general(mpc, mpc, dn, preferred_element_type=_F32)

    @pl.when(i == NB - 1)
    def _():
        glp_ref[...] = glp_s[...]
        gmp_ref[...] = gmp_s[...]


def _gram_call(t1, t2, item, s1, s2, si):
    vec = pl.BlockSpec((1, EMB), lambda i: (0, 0))
    return pl.pallas_call(
        _gram_body,
        grid=(NB,),
        in_specs=[
            pl.BlockSpec((RB, EMB), lambda i: (i, 0)),
            pl.BlockSpec((RB, EMB), lambda i: (i, 0)),
            pl.BlockSpec((RB, EMB), lambda i: (i, 0)),
            vec, vec, vec,
        ],
        out_specs=[
            pl.BlockSpec((EMB, EMB), lambda i: (0, 0)),
            pl.BlockSpec((EMB, EMB), lambda i: (0, 0)),
        ],
        out_shape=[
            jax.ShapeDtypeStruct((EMB, EMB), _F32),
            jax.ShapeDtypeStruct((EMB, EMB), _F32),
        ],
        scratch_shapes=[
            pltpu.VMEM((2, EMB), _F32),
            pltpu.VMEM((EMB, EMB), _F32),
            pltpu.VMEM((EMB, EMB), _F32),
        ],
    )(t1, t2, item, s1, s2, si)


def _combine_body(s1_ref, s2_ref, si_ref, glp_ref, gmp_ref, wl_ref, wm_ref,
                  g1_ref, be1_ref, g2_ref, be2_ref, wct_ref, wcb_ref, bcat_ref,
                  a_ref, b_ref, c_ref):
    ninv = 1.0 / N
    mu_lp = 0.5 * (s1_ref[...] + si_ref[...]) * ninv
    mu_mp = 0.5 * (s2_ref[...] - si_ref[...]) * ninv
    wl = wl_ref[...]
    wm = wm_ref[...]
    m1 = jnp.dot(mu_lp, wl, preferred_element_type=_F32)
    var1 = jnp.sum(jnp.dot(glp_ref[...], wl, preferred_element_type=_F32) * wl,
                   axis=0, keepdims=True) * ninv
    a1 = g1_ref[...] / jnp.sqrt(var1 + 1e-5)
    m2 = jnp.dot(mu_mp, wm, preferred_element_type=_F32)
    var2 = jnp.sum(jnp.dot(gmp_ref[...], wm, preferred_element_type=_F32) * wm,
                   axis=0, keepdims=True) * ninv
    a2 = g2_ref[...] / jnp.sqrt(var2 + 1e-5)
    wct = wct_ref[...]
    wcb = wcb_ref[...]
    a_ref[...] = 0.5 * jnp.dot(wl * a1, wct, preferred_element_type=_F32)
    b_ref[...] = 0.5 * jnp.dot(wm * a2, wcb, preferred_element_type=_F32)
    c_ref[...] = (jnp.dot(be1_ref[...] - m1 * a1, wct, preferred_element_type=_F32)
                  + jnp.dot(be2_ref[...] - m2 * a2, wcb, preferred_element_type=_F32)
                  + bcat_ref[...])


def _combine_call(s1, s2, si, glp, gmp, wl, wm, g1, be1, g2, be2, wct, wcb, bcat):
    return pl.pallas_call(
        _combine_body,
        out_shape=[
            jax.ShapeDtypeStruct((EMB, EMB), _F32),
            jax.ShapeDtypeStruct((EMB, EMB), _F32),
            jax.ShapeDtypeStruct((1, EMB), _F32),
        ],
    )(s1, s2, si, glp, gmp, wl, wm, g1, be1, g2, be2, wct, wcb, bcat)


_SC_INFO = plsc.get_sparse_core_info()
_NW = _SC_INFO.num_cores * _SC_INFO.num_subcores
_BPW = NGATHER // _NW
_CH = 96


def _sc_gather_body(t1_hbm, t2_hbm, it_hbm, idx_hbm, o1, o2, o3,
                    idx_v, rows_v, sem):
    wid = lax.axis_index("s") * _SC_INFO.num_cores + lax.axis_index("c")
    base = wid * _BPW
    for ci in range(_BPW // _CH):
        off = base + ci * _CH
        pltpu.sync_copy(idx_hbm.at[pl.ds(off, _CH)], idx_v)
        for tab, out in ((t1_hbm, o1), (t2_hbm, o2), (it_hbm, o3)):
            pltpu.async_copy(tab.at[idx_v], rows_v, sem).wait()
            pltpu.sync_copy(rows_v, out.at[pl.ds(off, _CH)])


_sc_gather = functools.partial(
    pl.kernel,
    mesh=plsc.VectorSubcoreMesh(core_axis_name="c", subcore_axis_name="s"),
    out_type=[jax.ShapeDtypeStruct((NGATHER, EMB), _F32)] * 3,
    scratch_types=[
        pltpu.VMEM((_CH,), jnp.int32),
        pltpu.VMEM((_CH, EMB), _F32),
        pltpu.SemaphoreType.DMA,
    ],
)(_sc_gather_body)


def _loss_body(t1g_ref, t2g_ref, ig_ref, a_ref, b_ref, c_ref, out_ref):
    ig = ig_ref[...]
    og = (jnp.dot(t1g_ref[...] + ig, a_ref[...], preferred_element_type=_F32)
          + jnp.dot(t2g_ref[...] - ig, b_ref[...], preferred_element_type=_F32)
          + c_ref[...])
    key = og[0:BSZ]
    pos = og[BSZ:2 * BSZ]
    ps = jnp.sum(key * pos, axis=1, keepdims=True)
    acc = jnp.zeros((1, 1), _F32)
    for k in range(NNEG):
        ns = jnp.sum(key * og[(2 + k) * BSZ:(3 + k) * BSZ], axis=1, keepdims=True)
        x = ps - ns
        sig = 1.0 / (1.0 + jnp.exp(-x))
        acc = acc + jnp.sum(jnp.log(sig + 1e-9))
    out_ref[...] = -acc / (BSZ * NNEG)


def _loss_call(t1g, t2g, ig, a, b, c):
    return pl.pallas_call(
        _loss_body,
        out_shape=jax.ShapeDtypeStruct((1, 1), _F32),
    )(t1g, t2g, ig, a, b, c)


def kernel(features, price, adj, train_set, W_cid2, b_cid2, W_cid3, b_cid3,
           emb_price, W_emb, b_emb, W_low, W_mid, g1, be1, g2, be2,
           W_cat, b_cat):
    pricei = price.reshape(N, 1)
    ep_pad = jnp.pad(emb_price, ((0, 128 - NBINS), (0, 0)))
    r = lambda v: v.reshape(1, EMB)

    item, s_it = _item_call(features, pricei, W_cid2, W_cid3, W_emb,
                            r(b_cid2), r(b_cid3), r(b_emb), ep_pad)
    t1, s1 = _spmm_call(adj, item)
    t2, s2 = _spmm_call(adj, t1)
    glp, gmp = _gram_call(t1, t2, item, s1, s2, s_it)
    a, b, c = _combine_call(s1, s2, s_it, glp, gmp, W_low, W_mid,
                            r(g1), r(be1), r(g2), r(be2),
                            W_cat[:EMB], W_cat[EMB:], r(b_cat))
    idx = train_set.T.reshape(-1)
    t1g, t2g, ig = _sc_gather(t1, t2, item, idx)
    loss = _loss_call(t1g, t2g, ig, a, b, c)
    return loss.reshape(())


# speedup vs baseline: 1.0818x; 1.0004x over previous
import functools

import jax
import jax.numpy as jnp
from jax import lax
from jax.experimental import pallas as pl
from jax.experimental.pallas import tpu as pltpu
from jax.experimental.pallas import tpu_sc as plsc

N = 10000
EMB = 128
CAT = 768
NBINS = 100
BSZ = 1024
NNEG = 4

RB = 400
NB = N // RB
NGATHER = BSZ * (2 + NNEG)

_F32 = jnp.float32


def _item_body(f_ref, p_ref, wc2_ref, wc3_ref, we_ref, bc2_ref, bc3_ref,
               be_ref, ep_ref, item_ref, s_ref, wf_s, pf_s, b_s, acc_s):
    i = pl.program_id(0)

    @pl.when(i == 0)
    def _():
        we1 = we_ref[0:EMB, :]
        we2 = we_ref[EMB:2 * EMB, :]
        we3 = we_ref[2 * EMB:3 * EMB, :]
        wf_s[0:CAT, :] = jnp.dot(wc2_ref[...], we1, preferred_element_type=_F32)
        wf_s[CAT:2 * CAT, :] = jnp.dot(wc3_ref[...], we2, preferred_element_type=_F32)
        pf_s[...] = jnp.dot(ep_ref[...], we3, preferred_element_type=_F32)
        b_s[...] = (jnp.dot(bc2_ref[...], we1, preferred_element_type=_F32)
                    + jnp.dot(bc3_ref[...], we2, preferred_element_type=_F32)
                    + be_ref[...])
        acc_s[...] = jnp.zeros_like(acc_s)

    onehot = (p_ref[...] == lax.broadcasted_iota(jnp.int32, (RB, 128), 1)).astype(_F32)
    x = jnp.dot(f_ref[...].astype(jnp.bfloat16), wf_s[...].astype(jnp.bfloat16),
                preferred_element_type=_F32)
    x = x + jnp.dot(onehot, pf_s[...], preferred_element_type=_F32) + b_s[...]
    it = jnp.maximum(x, 0.0)
    item_ref[...] = it
    acc_s[...] += jnp.sum(it, axis=0, keepdims=True)

    @pl.when(i == NB - 1)
    def _():
        s_ref[...] = acc_s[...]


def _item_call(features, pricef, wc2, wc3, we, bc2, bc3, be, ep_pad):
    return pl.pallas_call(
        _item_body,
        grid=(NB,),
        in_specs=[
            pl.BlockSpec((RB, 2 * CAT), lambda i: (i, 0)),
            pl.BlockSpec((RB, 1), lambda i: (i, 0)),
            pl.BlockSpec((CAT, EMB), lambda i: (0, 0)),
            pl.BlockSpec((CAT, EMB), lambda i: (0, 0)),
            pl.BlockSpec((3 * EMB, EMB), lambda i: (0, 0)),
            pl.BlockSpec((1, EMB), lambda i: (0, 0)),
            pl.BlockSpec((1, EMB), lambda i: (0, 0)),
            pl.BlockSpec((1, EMB), lambda i: (0, 0)),
            pl.BlockSpec((128, EMB), lambda i: (0, 0)),
        ],
        out_specs=[
            pl.BlockSpec((RB, EMB), lambda i: (i, 0)),
            pl.BlockSpec((1, EMB), lambda i: (0, 0)),
        ],
        out_shape=[
            jax.ShapeDtypeStruct((N, EMB), _F32),
            jax.ShapeDtypeStruct((1, EMB), _F32),
        ],
        scratch_shapes=[
            pltpu.VMEM((2 * CAT, EMB), _F32),
            pltpu.VMEM((128, EMB), _F32),
            pltpu.VMEM((1, EMB), _F32),
            pltpu.VMEM((1, EMB), _F32),
        ],
    )(features, pricef, wc2, wc3, we, bc2, bc3, be, ep_pad)


def _spmm_body(adj_ref, x_ref, t_ref, s_ref, acc_s):
    i = pl.program_id(0)

    @pl.when(i == 0)
    def _():
        acc_s[...] = jnp.zeros_like(acc_s)

    t = jnp.dot(adj_ref[...].astype(jnp.bfloat16), x_ref[...].astype(jnp.bfloat16),
                preferred_element_type=_F32)
    t_ref[...] = t
    acc_s[...] += jnp.sum(t, axis=0, keepdims=True)

    @pl.when(i == NB - 1)
    def _():
        s_ref[...] = acc_s[...]


def _spmm_call(adj, x):
    return pl.pallas_call(
        _spmm_body,
        grid=(NB,),
        in_specs=[
            pl.BlockSpec((RB, N), lambda i: (i, 0)),
            pl.BlockSpec((N, EMB), lambda i: (0, 0)),
        ],
        out_specs=[
            pl.BlockSpec((RB, EMB), lambda i: (i, 0)),
            pl.BlockSpec((1, EMB), lambda i: (0, 0)),
        ],
        out_shape=[
            jax.ShapeDtypeStruct((N, EMB), _F32),
            jax.ShapeDtypeStruct((1, EMB), _F32),
        ],
        scratch_shapes=[pltpu.VMEM((1, EMB), _F32)],
    )(adj, x)


def _gram_body(t1_ref, t2_ref, it_ref, s1_ref, s2_ref, si_ref,
               glp_ref, gmp_ref, mu_s, glp_s, gmp_s):
    i = pl.program_id(0)

    @pl.when(i == 0)
    def _():
        ninv = 1.0 / N
        mu_s[0:1, :] = 0.5 * (s1_ref[...] + si_ref[...]) * ninv
        mu_s[1:2, :] = 0.5 * (s2_ref[...] - si_ref[...]) * ninv
        glp_s[...] = jnp.zeros_like(glp_s)
        gmp_s[...] = jnp.zeros_like(gmp_s)

    t1 = t1_ref[...]
    t2 = t2_ref[...]
    it = it_ref[...]
    lpc = 0.5 * (t1 + it) - mu_s[0:1, :]
    mpc = 0.5 * (t2 - it) - mu_s[1:2, :]
    dn = (((0,), (0,)), ((), ()))
    glp_s[...] += lax.dot_general(lpc, lpc, dn, preferred_element_type=_F32)
    gmp_s[...] += lax.dot_general(mpc, mpc, dn, preferred_element_type=_F32)

    @pl.when(i == NB - 1)
    def _():
        glp_ref[...] = glp_s[...]
        gmp_ref[...] = gmp_s[...]


def _gram_call(t1, t2, item, s1, s2, si):
    vec = pl.BlockSpec((1, EMB), lambda i: (0, 0))
    return pl.pallas_call(
        _gram_body,
        grid=(NB,),
        in_specs=[
            pl.BlockSpec((RB, EMB), lambda i: (i, 0)),
            pl.BlockSpec((RB, EMB), lambda i: (i, 0)),
            pl.BlockSpec((RB, EMB), lambda i: (i, 0)),
            vec, vec, vec,
        ],
        out_specs=[
            pl.BlockSpec((EMB, EMB), lambda i: (0, 0)),
            pl.BlockSpec((EMB, EMB), lambda i: (0, 0)),
        ],
        out_shape=[
            jax.ShapeDtypeStruct((EMB, EMB), _F32),
            jax.ShapeDtypeStruct((EMB, EMB), _F32),
        ],
        scratch_shapes=[
            pltpu.VMEM((2, EMB), _F32),
            pltpu.VMEM((EMB, EMB), _F32),
            pltpu.VMEM((EMB, EMB), _F32),
        ],
    )(t1, t2, item, s1, s2, si)


def _combine_body(s1_ref, s2_ref, si_ref, glp_ref, gmp_ref, wl_ref, wm_ref,
                  g1_ref, be1_ref, g2_ref, be2_ref, wct_ref, wcb_ref, bcat_ref,
                  a_ref, b_ref, c_ref):
    ninv = 1.0 / N
    mu_lp = 0.5 * (s1_ref[...] + si_ref[...]) * ninv
    mu_mp = 0.5 * (s2_ref[...] - si_ref[...]) * ninv
    wl = wl_ref[...]
    wm = wm_ref[...]
    m1 = jnp.dot(mu_lp, wl, preferred_element_type=_F32)
    var1 = jnp.sum(jnp.dot(glp_ref[...], wl, preferred_element_type=_F32) * wl,
                   axis=0, keepdims=True) * ninv
    a1 = g1_ref[...] / jnp.sqrt(var1 + 1e-5)
    m2 = jnp.dot(mu_mp, wm, preferred_element_type=_F32)
    var2 = jnp.sum(jnp.dot(gmp_ref[...], wm, preferred_element_type=_F32) * wm,
                   axis=0, keepdims=True) * ninv
    a2 = g2_ref[...] / jnp.sqrt(var2 + 1e-5)
    wct = wct_ref[...]
    wcb = wcb_ref[...]
    a_ref[...] = 0.5 * jnp.dot(wl * a1, wct, preferred_element_type=_F32)
    b_ref[...] = 0.5 * jnp.dot(wm * a2, wcb, preferred_element_type=_F32)
    c_ref[...] = (jnp.dot(be1_ref[...] - m1 * a1, wct, preferred_element_type=_F32)
                  + jnp.dot(be2_ref[...] - m2 * a2, wcb, preferred_element_type=_F32)
                  + bcat_ref[...])


def _combine_call(s1, s2, si, glp, gmp, wl, wm, g1, be1, g2, be2, wct, wcb, bcat):
    return pl.pallas_call(
        _combine_body,
        out_shape=[
            jax.ShapeDtypeStruct((EMB, EMB), _F32),
            jax.ShapeDtypeStruct((EMB, EMB), _F32),
            jax.ShapeDtypeStruct((1, EMB), _F32),
        ],
    )(s1, s2, si, glp, gmp, wl, wm, g1, be1, g2, be2, wct, wcb, bcat)


_SC_INFO = plsc.get_sparse_core_info()
_NW = _SC_INFO.num_cores * _SC_INFO.num_subcores
_BPW = NGATHER // _NW
_CH = 96


def _sc_gather_body(t1_hbm, t2_hbm, it_hbm, idx_hbm, o1, o2, o3,
                    idx_v, rows_v, sem):
    wid = lax.axis_index("s") * _SC_INFO.num_cores + lax.axis_index("c")
    base = wid * _BPW
    for ci in range(_BPW // _CH):
        off = base + ci * _CH
        pltpu.sync_copy(idx_hbm.at[pl.ds(off, _CH)], idx_v)
        for tab, out in ((t1_hbm, o1), (t2_hbm, o2), (it_hbm, o3)):
            pltpu.async_copy(tab.at[idx_v], rows_v, sem).wait()
            pltpu.sync_copy(rows_v, out.at[pl.ds(off, _CH)])


_sc_gather = functools.partial(
    pl.kernel,
    mesh=plsc.VectorSubcoreMesh(core_axis_name="c", subcore_axis_name="s"),
    out_type=[jax.ShapeDtypeStruct((NGATHER, EMB), _F32)] * 3,
    scratch_types=[
        pltpu.VMEM((_CH,), jnp.int32),
        pltpu.VMEM((_CH, EMB), _F32),
        pltpu.SemaphoreType.DMA,
    ],
)(_sc_gather_body)


def _loss_body(t1g_ref, t2g_ref, ig_ref, a_ref, b_ref, c_ref, out_ref):
    ig = ig_ref[...]
    og = (jnp.dot(t1g_ref[...] + ig, a_ref[...], preferred_element_type=_F32)
          + jnp.dot(t2g_ref[...] - ig, b_ref[...], preferred_element_type=_F32)
          + c_ref[...])
    key = og[0:BSZ]
    pos = og[BSZ:2 * BSZ]
    ps = jnp.sum(key * pos, axis=1, keepdims=True)
    acc = jnp.zeros((1, 1), _F32)
    for k in range(NNEG):
        ns = jnp.sum(key * og[(2 + k) * BSZ:(3 + k) * BSZ], axis=1, keepdims=True)
        x = ps - ns
        sig = 1.0 / (1.0 + jnp.exp(-x))
        acc = acc + jnp.sum(jnp.log(sig + 1e-9))
    out_ref[...] = -acc / (BSZ * NNEG)


def _loss_call(t1g, t2g, ig, a, b, c):
    return pl.pallas_call(
        _loss_body,
        out_shape=jax.ShapeDtypeStruct((1, 1), _F32),
    )(t1g, t2g, ig, a, b, c)


def kernel(features, price, adj, train_set, W_cid2, b_cid2, W_cid3, b_cid3,
           emb_price, W_emb, b_emb, W_low, W_mid, g1, be1, g2, be2,
           W_cat, b_cat):
    pricei = price.reshape(N, 1)
    ep_pad = jnp.pad(emb_price, ((0, 128 - NBINS), (0, 0)))
    r = lambda v: v.reshape(1, EMB)

    item, s_it = _item_call(features, pricei, W_cid2, W_cid3, W_emb,
                            r(b_cid2), r(b_cid3), r(b_emb), ep_pad)
    t1, s1 = _spmm_call(adj, item)
    t2, s2 = _spmm_call(adj, t1)
    glp, gmp = _gram_call(t1, t2, item, s1, s2, s_it)
    a, b, c = _combine_call(s1, s2, s_it, glp, gmp, W_low, W_mid,
                            r(g1), r(be1), r(g2), r(be2),
                            W_cat[:EMB], W_cat[EMB:], r(b_cat))
    idx = train_set.T.reshape(-1)
    t1g, t2g, ig = _sc_gather(t1, t2, item, idx)
    loss = _loss_call(t1g, t2g, ig, a, b, c)
    return loss.reshape(())
